# baseline (speedup 1.0000x reference)
import functools

import jax
import jax.numpy as jnp
from jax import lax
from jax.experimental import pallas as pl
from jax.experimental.pallas import tpu as pltpu
from jax.experimental.pallas import tpu_sc as plsc

N = 16384
NC = 2
NS = 16
NW = NC * NS
CH = 2048
L = 16
SEG = N // NS
D = 4096
DT = D // NS

_MESH = plsc.VectorSubcoreMesh(
    core_axis_name="c", subcore_axis_name="s", num_cores=NC,
    num_subcores=NS)


def _build_layer(epw, decide):
  nch = epw // CH

  out_type = [jax.ShapeDtypeStruct((NC, N), jnp.float32)]
  scratch = [
      pltpu.VMEM((N,), jnp.float32),
      pltpu.VMEM((N,), jnp.float32),
      pltpu.VMEM((N,), jnp.float32),
      pltpu.VMEM((CH,), jnp.int32),
      pltpu.VMEM((CH,), jnp.float32),
      pltpu.VMEM((CH,), jnp.int32),
      pltpu.VMEM((CH,), jnp.float32),
      pltpu.VMEM((CH,), jnp.int32),
      pltpu.VMEM((CH,), jnp.float32),
      pltpu.VMEM((NS, SEG), jnp.float32),
      pltpu.VMEM((SEG,), jnp.float32),
      pltpu.VMEM_SHARED((NS, N), jnp.float32),
      pltpu.SemaphoreType.DMA,
      pltpu.SemaphoreType.DMA,
      pltpu.SemaphoreType.DMA,
  ]
  if decide:
    out_type.append(jax.ShapeDtypeStruct((NC, L), jnp.float32))
    scratch += [
        pltpu.VMEM((2, 128), jnp.int32),
        pltpu.VMEM((DT,), jnp.float32),
        pltpu.VMEM((DT,), jnp.float32),
        pltpu.VMEM((2, 128), jnp.float32),
        pltpu.VMEM((L,), jnp.float32),
        pltpu.VMEM((NS, 128), jnp.float32),
        pltpu.VMEM_SHARED((N,), jnp.float32),
        pltpu.VMEM_SHARED((NS, 128), jnp.float32),
    ]

  def layer_body(*refs):
    if decide:
      (pin, pk_hbm, ev_hbm, dcol, dval, fcw, pout, out2, x_v, acc_v,
       acc2_v, pk_buf0, ev_buf0, pk_buf1, ev_buf1, pk_buf2, ev_buf2,
       red_v, res_v, s_all, sem0, sem1, sem2, idx_v, dv_v, fw_v, g_v,
       part_v, sum_v, s_red, s_part) = refs
    else:
      (pin, pk_hbm, ev_hbm, pout, x_v, acc_v, acc2_v, pk_buf0, ev_buf0,
       pk_buf1, ev_buf1, pk_buf2, ev_buf2, red_v, res_v, s_all, sem0,
       sem1, sem2) = refs
    c = lax.axis_index("c")
    s = lax.axis_index("s")
    w = c * NS + s
    bufs = ((pk_buf0, ev_buf0, sem0), (pk_buf1, ev_buf1, sem1),
            (pk_buf2, ev_buf2, sem2))

    def copies(ci, slot):
      off = w * epw + ci * CH
      pk_b, ev_b, sem = bufs[slot]
      return (pltpu.make_async_copy(pk_hbm.at[pl.ds(off, CH)], pk_b, sem),
              pltpu.make_async_copy(ev_hbm.at[pl.ds(off, CH)], ev_b, sem))

    def start(ci, slot):
      for cp in copies(ci, slot):
        cp.start()

    def drain(ci, slot):
      for cp in copies(ci, slot):
        cp.wait()

    start(0, 0)
    start(1, 1)
    xcp = (pltpu.make_async_copy(pin.at[0], x_v, sem2),
           pltpu.make_async_copy(pin.at[1], acc_v, sem2))
    for cp in xcp:
      cp.start()
    for cp in xcp:
      cp.wait()

    @plsc.parallel_loop(0, N // L, unroll=4)
    def _(i):
      x_v[pl.ds(i * L, L)] = x_v[pl.ds(i * L, L)] + acc_v[pl.ds(i * L, L)]
      acc_v[pl.ds(i * L, L)] = jnp.zeros((L,), jnp.float32)
      acc2_v[pl.ds(i * L, L)] = jnp.zeros((L,), jnp.float32)

    def compute(slot):
      pk_b, ev_b, _ = bufs[slot]

      @plsc.parallel_loop(0, CH // (2 * L), unroll=8)
      def _(i):
        for h, acc_t in ((0, acc_v), (1, acc2_v)):
          pk16 = pk_b[pl.ds((2 * i + h) * L, L)]
          ev16 = ev_b[pl.ds((2 * i + h) * L, L)]
          col = pk16 & (N - 1)
          row = pk16 >> 14
          xg = plsc.load_gather(x_v, [col])
          plsc.addupdate_scatter(acc_t, [row], ev16 * xg)

    start(2, 2)

    def tri(ti, _):
      ci = ti * 3
      for b in range(3):
        drain(ci + b, b)
        compute(b)

        @pl.when(ci + b + 3 < nch)
        def _():
          start(ci + b + 3, b)
      return 0
    lax.fori_loop(0, nch // 3, tri, 0)

    @plsc.parallel_loop(0, N // L, unroll=4)
    def _(i):
      acc_v[pl.ds(i * L, L)] = (
          acc_v[pl.ds(i * L, L)] + acc2_v[pl.ds(i * L, L)])
    pltpu.sync_copy(acc_v, s_all.at[s])
    plsc.subcore_barrier()
    rcp = [pltpu.make_async_copy(s_all.at[j, pl.ds(s * SEG, SEG)],
                                 red_v.at[j], sem2) for j in range(NS)]
    for cp in rcp:
      cp.start()
    for cp in rcp:
      cp.wait()

    @plsc.parallel_loop(0, SEG // L, unroll=2)
    def _(i):
      tot = red_v[0, pl.ds(i * L, L)]
      for j in range(1, NS):
        tot = tot + red_v[j, pl.ds(i * L, L)]
      res_v[pl.ds(i * L, L)] = tot
    pltpu.sync_copy(res_v, pout.at[c, pl.ds(s * SEG, SEG)])

    if decide:
      pltpu.sync_copy(res_v, s_red.at[pl.ds(s * SEG, SEG)])
      base = s * DT
      for k in range(2):
        pltpu.sync_copy(dcol.at[pl.ds(base + k * 128, 128)], idx_v.at[k])
      pltpu.sync_copy(dval.at[pl.ds(base, DT)], dv_v)
      pltpu.sync_copy(fcw.at[pl.ds(base, DT)], fw_v)
      @plsc.parallel_loop(0, DT // L, unroll=2)
      def _(i):
        dv_v[pl.ds(i * L, L)] = dv_v[pl.ds(i * L, L)] * fw_v[pl.ds(i * L, L)]
      plsc.subcore_barrier()
      for k in range(2):
        pltpu.async_copy(s_red.at[idx_v.at[k]], g_v.at[k], sem0).wait()
      acc = jnp.zeros((L,), jnp.float32)
      for i in range(DT // L):
        g16 = g_v[i // 8, pl.ds((i % 8) * L, L)]
        acc = acc + g16 * dv_v[pl.ds(i * L, L)]
      part_v[...] = acc
      pltpu.sync_copy(part_v, s_part.at[s, pl.ds(0, L)])
      plsc.subcore_barrier()

      @pl.when(s == 0)
      def _():
        pltpu.sync_copy(s_part, sum_v)
        tot = sum_v[0, pl.ds(0, L)]
        for j in range(1, NS):
          tot = tot + sum_v[j, pl.ds(0, L)]
        part_v[...] = jnp.broadcast_to(jnp.sum(tot), (L,))
        pltpu.sync_copy(part_v, out2.at[c])

  return functools.partial(
      pl.kernel,
      out_type=out_type if decide else out_type[0],
      mesh=_MESH,
      compiler_params=pltpu.CompilerParams(needs_layout_passes=False),
      scratch_types=scratch,
  )(layer_body)


def kernel(x, adj_vals, dec_vals, fc_w, fc_b, adj_rows, adj_cols, dec_rows,
           dec_cols):
  nnz = adj_vals.shape[0]
  layers = 4

  nch = -(-nnz // (NW * CH))
  nch += (-nch) % 3
  epw = nch * CH
  pad = NW * epw - nnz

  pk = (adj_rows << 14) | adj_cols
  pad_pk = (jnp.arange(pad, dtype=jnp.int32) % N) << 14
  pk = jnp.concatenate([pk, pad_pk])
  ev = jnp.concatenate([adj_vals, jnp.zeros((pad,), jnp.float32)])

  p = jnp.stack([x[:, 0], jnp.zeros((N,), jnp.float32)])

  layer = _build_layer(epw, decide=False)
  for _ in range(layers - 1):
    p = layer(p, pk, ev)
  layer_dec = _build_layer(epw, decide=True)
  p, o2 = layer_dec(p, pk, ev, dec_cols, dec_vals, fc_w.reshape(-1))
  return o2[0, 0] + o2[1, 0] + fc_b


# speedup vs baseline: 1.0446x; 1.0446x over previous
import functools

import jax
import jax.numpy as jnp
from jax import lax
from jax.experimental import pallas as pl
from jax.experimental.pallas import tpu as pltpu
from jax.experimental.pallas import tpu_sc as plsc

N = 16384
NC = 2
NS = 16
NW = NC * NS
CH = 2048
L = 16
SEG = N // NS
D = 4096
DT = D // NS

_MESH = plsc.VectorSubcoreMesh(
    core_axis_name="c", subcore_axis_name="s", num_cores=NC,
    num_subcores=NS)


def _build_layer(epw, decide):
  nch = epw // CH

  out_type = [jax.ShapeDtypeStruct((NC, N), jnp.float32)]
  scratch = [
      pltpu.VMEM((N,), jnp.float32),
      pltpu.VMEM((N,), jnp.float32),
      pltpu.VMEM((CH,), jnp.int32),
      pltpu.VMEM((CH,), jnp.float32),
      pltpu.VMEM((CH,), jnp.int32),
      pltpu.VMEM((CH,), jnp.float32),
      pltpu.VMEM((CH,), jnp.int32),
      pltpu.VMEM((CH,), jnp.float32),
      pltpu.VMEM((NS, SEG), jnp.float32),
      pltpu.VMEM((SEG,), jnp.float32),
      pltpu.VMEM_SHARED((NS, N), jnp.float32),
      pltpu.SemaphoreType.DMA,
      pltpu.SemaphoreType.DMA,
      pltpu.SemaphoreType.DMA,
  ]
  if decide:
    out_type.append(jax.ShapeDtypeStruct((NC, L), jnp.float32))
    scratch += [
        pltpu.VMEM((2, 128), jnp.int32),
        pltpu.VMEM((DT,), jnp.float32),
        pltpu.VMEM((DT,), jnp.float32),
        pltpu.VMEM((2, 128), jnp.float32),
        pltpu.VMEM((L,), jnp.float32),
        pltpu.VMEM((NS, 128), jnp.float32),
        pltpu.VMEM_SHARED((N,), jnp.float32),
        pltpu.VMEM_SHARED((NS, 128), jnp.float32),
    ]

  def layer_body(*refs):
    if decide:
      (pin, pk_hbm, ev_hbm, dcol, dval, fcw, pout, out2, x_v, acc_v,
       pk_buf0, ev_buf0, pk_buf1, ev_buf1, pk_buf2, ev_buf2, red_v, res_v,
       s_all, sem0, sem1, sem2, idx_v, dv_v, fw_v, g_v, part_v, sum_v,
       s_red, s_part) = refs
    else:
      (pin, pk_hbm, ev_hbm, pout, x_v, acc_v, pk_buf0, ev_buf0, pk_buf1,
       ev_buf1, pk_buf2, ev_buf2, red_v, res_v, s_all, sem0, sem1,
       sem2) = refs
    c = lax.axis_index("c")
    s = lax.axis_index("s")
    w = c * NS + s
    bufs = ((pk_buf0, ev_buf0, sem0), (pk_buf1, ev_buf1, sem1),
            (pk_buf2, ev_buf2, sem2))

    def copies(ci, slot):
      off = w * epw + ci * CH
      pk_b, ev_b, sem = bufs[slot]
      return (pltpu.make_async_copy(pk_hbm.at[pl.ds(off, CH)], pk_b, sem),
              pltpu.make_async_copy(ev_hbm.at[pl.ds(off, CH)], ev_b, sem))

    def start(ci, slot):
      for cp in copies(ci, slot):
        cp.start()

    def drain(ci, slot):
      for cp in copies(ci, slot):
        cp.wait()

    start(0, 0)
    start(1, 1)
    xcp = (pltpu.make_async_copy(pin.at[0], x_v, sem2),
           pltpu.make_async_copy(pin.at[1], acc_v, sem2))
    for cp in xcp:
      cp.start()
    for cp in xcp:
      cp.wait()

    @plsc.parallel_loop(0, N // L, unroll=4)
    def _(i):
      x_v[pl.ds(i * L, L)] = x_v[pl.ds(i * L, L)] + acc_v[pl.ds(i * L, L)]
      acc_v[pl.ds(i * L, L)] = jnp.zeros((L,), jnp.float32)

    def compute(slot):
      pk_b, ev_b, _ = bufs[slot]

      @plsc.parallel_loop(0, CH // L, unroll=16)
      def _(i):
        pk16 = pk_b[pl.ds(i * L, L)]
        ev16 = ev_b[pl.ds(i * L, L)]
        col = pk16 & (N - 1)
        row = pk16 >> 14
        xg = plsc.load_gather(x_v, [col])
        plsc.addupdate_scatter(acc_v, [row], ev16 * xg)

    start(2, 2)

    def tri(ti, _):
      ci = ti * 3
      for b in range(3):
        drain(ci + b, b)
        compute(b)

        @pl.when(ci + b + 3 < nch)
        def _():
          start(ci + b + 3, b)
      return 0
    lax.fori_loop(0, nch // 3, tri, 0)

    pltpu.sync_copy(acc_v, s_all.at[s])
    plsc.subcore_barrier()
    rcp = [pltpu.make_async_copy(s_all.at[j, pl.ds(s * SEG, SEG)],
                                 red_v.at[j], sem2) for j in range(NS)]
    for cp in rcp:
      cp.start()
    for cp in rcp:
      cp.wait()

    @plsc.parallel_loop(0, SEG // L, unroll=2)
    def _(i):
      tot = red_v[0, pl.ds(i * L, L)]
      for j in range(1, NS):
        tot = tot + red_v[j, pl.ds(i * L, L)]
      res_v[pl.ds(i * L, L)] = tot
    pltpu.sync_copy(res_v, pout.at[c, pl.ds(s * SEG, SEG)])

    if decide:
      pltpu.sync_copy(res_v, s_red.at[pl.ds(s * SEG, SEG)])
      base = s * DT
      for k in range(2):
        pltpu.sync_copy(dcol.at[pl.ds(base + k * 128, 128)], idx_v.at[k])
      pltpu.sync_copy(dval.at[pl.ds(base, DT)], dv_v)
      pltpu.sync_copy(fcw.at[pl.ds(base, DT)], fw_v)
      @plsc.parallel_loop(0, DT // L, unroll=2)
      def _(i):
        dv_v[pl.ds(i * L, L)] = dv_v[pl.ds(i * L, L)] * fw_v[pl.ds(i * L, L)]
      plsc.subcore_barrier()
      for k in range(2):
        pltpu.async_copy(s_red.at[idx_v.at[k]], g_v.at[k], sem0).wait()
      acc = jnp.zeros((L,), jnp.float32)
      for i in range(DT // L):
        g16 = g_v[i // 8, pl.ds((i % 8) * L, L)]
        acc = acc + g16 * dv_v[pl.ds(i * L, L)]
      part_v[...] = acc
      pltpu.sync_copy(part_v, s_part.at[s, pl.ds(0, L)])
      plsc.subcore_barrier()

      @pl.when(s == 0)
      def _():
        pltpu.sync_copy(s_part, sum_v)
        tot = sum_v[0, pl.ds(0, L)]
        for j in range(1, NS):
          tot = tot + sum_v[j, pl.ds(0, L)]
        part_v[...] = jnp.broadcast_to(jnp.sum(tot), (L,))
        pltpu.sync_copy(part_v, out2.at[c])

  return functools.partial(
      pl.kernel,
      out_type=out_type if decide else out_type[0],
      mesh=_MESH,
      compiler_params=pltpu.CompilerParams(needs_layout_passes=False),
      scratch_types=scratch,
  )(layer_body)


def kernel(x, adj_vals, dec_vals, fc_w, fc_b, adj_rows, adj_cols, dec_rows,
           dec_cols):
  nnz = adj_vals.shape[0]
  layers = 4

  nch = -(-nnz // (NW * CH))
  nch += (-nch) % 3
  epw = nch * CH
  pad = NW * epw - nnz

  pk = (adj_rows << 14) | adj_cols
  pad_pk = (jnp.arange(pad, dtype=jnp.int32) % N) << 14
  pk = jnp.concatenate([pk, pad_pk])
  ev = jnp.concatenate([adj_vals, jnp.zeros((pad,), jnp.float32)])

  p = jnp.stack([x[:, 0], jnp.zeros((N,), jnp.float32)])

  layer = _build_layer(epw, decide=False)
  for _ in range(layers - 1):
    p = layer(p, pk, ev)
  layer_dec = _build_layer(epw, decide=True)
  p, o2 = layer_dec(p, pk, ev, dec_cols, dec_vals, fc_w.reshape(-1))
  return o2[0, 0] + o2[1, 0] + fc_b
